# Initial kernel scaffold; baseline (speedup 1.0000x reference)
import jax
import jax.numpy as jnp
from jax.experimental import pallas as pl


def kernel(x, edge_index, W_mu1, b_mu1, W_mu2, b_mu2, W_mu3, b_mu3, W_lg1, b_lg1, W_lg2, b_lg2, W_lg3, b_lg3):
    raise NotImplementedError("write your pallas kernel here")


# speedup vs baseline: 13.5969x; 13.5969x over previous
import functools

import jax
import jax.numpy as jnp
from jax import lax
from jax.experimental import pallas as pl
from jax.experimental.pallas import tpu as pltpu
from jax.experimental.pallas import tpu_sc as plsc

N = 10000
E = 320000
CHUNK = 128
R = E // CHUNK
NC, NS = 2, 16
NW = NC * NS
ROWS_BASE = R // NW
ROWS_EXTRA = R - ROWS_BASE * NW
NPS = 640
NPAD = NPS * NS

BN = 2000


def _mesh():
    return plsc.VectorSubcoreMesh(core_axis_name="c", subcore_axis_name="s")


def _worker(c, s):
    wid = c * NS + s
    start = ROWS_BASE * wid + jnp.minimum(wid, ROWS_EXTRA)
    n = ROWS_BASE + (wid < ROWS_EXTRA).astype(jnp.int32)
    return start, n


def _deg_body(dstr, out, zbuf, ones_v, idx_d, acc):
    c = lax.axis_index("c")
    s = lax.axis_index("s")
    zero16 = jnp.zeros((16,), jnp.float32)
    one16 = jnp.ones((16,), jnp.float32)
    for i in range(40):
        zbuf[pl.ds(i * 16, 16)] = zero16
    for i in range(8):
        ones_v[pl.ds(i * 16, 16)] = one16
    pltpu.sync_copy(zbuf, acc.at[pl.ds(s * 640, 640)])
    plsc.subcore_barrier()
    start, n = _worker(c, s)

    def body(r, carry):
        pltpu.sync_copy(dstr.at[r], idx_d)
        pltpu.sync_copy(ones_v, acc.at[idx_d], add=True)
        return carry

    lax.fori_loop(start, start + n, body, 0)
    plsc.subcore_barrier()
    pltpu.sync_copy(acc.at[pl.ds(s * 640, 640)], out.at[c, pl.ds(s * 640, 640)])


def _deg_call(dst2d):
    return pl.kernel(
        _deg_body,
        out_type=jax.ShapeDtypeStruct((NC, NPAD), jnp.float32),
        mesh=_mesh(),
        scratch_types=[
            pltpu.VMEM((640,), jnp.float32),
            pltpu.VMEM((CHUNK,), jnp.float32),
            pltpu.VMEM((CHUNK,), jnp.int32),
            pltpu.VMEM_SHARED((NPAD,), jnp.float32),
        ],
    )(dst2d)


def _seg_body(W, g, srcr, dstr, out, idx_s, idx_d, rows, acc, sem):
    c = lax.axis_index("c")
    s = lax.axis_index("s")
    zero16 = jnp.zeros((16,), jnp.float32)

    def zrow(r, carry):
        for j in range(W // 16):
            rows[r, pl.ds(j * 16, 16)] = zero16
        return carry

    lax.fori_loop(0, CHUNK, zrow, 0)
    base = s * NPS
    for k in range(NPS // CHUNK):
        pltpu.sync_copy(rows, acc.at[pl.ds(base + CHUNK * k, CHUNK), :])
    plsc.subcore_barrier()
    start, n = _worker(c, s)

    def body(r, carry):
        pltpu.sync_copy(srcr.at[r], idx_s)
        pltpu.sync_copy(dstr.at[r], idx_d)
        pltpu.async_copy(g.at[idx_s], rows, sem).wait()
        pltpu.sync_copy(rows, acc.at[idx_d], add=True)
        return carry

    lax.fori_loop(start, start + n, body, 0)
    plsc.subcore_barrier()
    pltpu.sync_copy(acc.at[pl.ds(base, NPS), :], out.at[c, pl.ds(base, NPS), :])


def _seg_call(g, src2d, dst2d):
    W = g.shape[1]
    return pl.kernel(
        functools.partial(_seg_body, W),
        out_type=jax.ShapeDtypeStruct((NC, NPAD, W), jnp.float32),
        mesh=_mesh(),
        scratch_types=[
            pltpu.VMEM((CHUNK,), jnp.int32),
            pltpu.VMEM((CHUNK,), jnp.int32),
            pltpu.VMEM((CHUNK, W), jnp.float32),
            pltpu.VMEM_SHARED((NPAD, W), jnp.float32),
            pltpu.SemaphoreType.DMA,
        ],
        compiler_params=pltpu.CompilerParams(use_tc_tiling_on_sc=False),
    )(g, src2d, dst2d)


def _dec_body(z, srcr, dstr, adj, idx_s, idx_d, zs, zd, abuf, sem):
    c = lax.axis_index("c")
    s = lax.axis_index("s")
    start, n = _worker(c, s)

    def body(r, carry):
        pltpu.sync_copy(srcr.at[r], idx_s)
        pltpu.sync_copy(dstr.at[r], idx_d)
        pltpu.async_copy(z.at[idx_s], zs, sem).wait()
        pltpu.async_copy(z.at[idx_d], zd, sem).wait()
        for e0 in range(0, CHUNK, 16):
            ridx = lax.iota(jnp.int32, 16) + e0
            acc = jnp.zeros((16,), jnp.float32)
            for j in range(32):
                col = jnp.full((16,), j, jnp.int32)
                a = plsc.load_gather(zs, [ridx, col])
                b = plsc.load_gather(zd, [ridx, col])
                acc = acc + a * b
            sig = 1.0 / (1.0 + jnp.exp(-acc))
            abuf[pl.ds(e0, 16)] = sig
        pltpu.sync_copy(abuf, adj.at[r])
        return carry

    lax.fori_loop(start, start + n, body, 0)


def _dec_call(z, src2d, dst2d):
    return pl.kernel(
        _dec_body,
        out_type=jax.ShapeDtypeStruct((R, CHUNK), jnp.float32),
        mesh=_mesh(),
        scratch_types=[
            pltpu.VMEM((CHUNK,), jnp.int32),
            pltpu.VMEM((CHUNK,), jnp.int32),
            pltpu.VMEM((CHUNK, 32), jnp.float32),
            pltpu.VMEM((CHUNK, 32), jnp.float32),
            pltpu.VMEM((CHUNK,), jnp.float32),
            pltpu.SemaphoreType.DMA,
        ],
        compiler_params=pltpu.CompilerParams(use_tc_tiling_on_sc=False,
                                             needs_layout_passes=False),
    )(z, src2d, dst2d)


def _row_spec(w):
    return pl.BlockSpec((BN, w), lambda i: (i, 0))


def _part_spec(w):
    return pl.BlockSpec((NC, BN, w), lambda i: (0, i, 0))


def _full_spec(a, b):
    return pl.BlockSpec((a, b), lambda i: (0, 0))


def _scale0_body(degp_ref, x_ref, dinv_ref, g0_ref):
    deg = degp_ref[0] + degp_ref[1] + 1.0
    dinv = lax.rsqrt(deg)
    dinv_ref[...] = dinv
    g0_ref[...] = x_ref[...] * dinv


def _scale0_call(degp, x):
    return pl.pallas_call(
        _scale0_body,
        grid=(N // BN,),
        in_specs=[_part_spec(1), _row_spec(128)],
        out_specs=[_row_spec(1), _row_spec(128)],
        out_shape=[jax.ShapeDtypeStruct((N, 1), jnp.float32),
                   jax.ShapeDtypeStruct((N, 128), jnp.float32)],
    )(degp, x)


def _dense1_body(s1_ref, g0_ref, dinv_ref, w_ref, b_ref, g1_ref):
    dinv = dinv_ref[...]
    a1 = (s1_ref[0] + s1_ref[1] + g0_ref[...]) * dinv
    h1 = jax.nn.relu(jnp.dot(a1, w_ref[...],
                             preferred_element_type=jnp.float32) + b_ref[...])
    g1_ref[...] = h1 * dinv


def _dense1_call(s1, g0, dinv, Wc1, bc1):
    return pl.pallas_call(
        _dense1_body,
        grid=(N // BN,),
        in_specs=[_part_spec(128), _row_spec(128), _row_spec(1),
                  _full_spec(128, 128), _full_spec(1, 128)],
        out_specs=_row_spec(128),
        out_shape=jax.ShapeDtypeStruct((N, 128), jnp.float32),
    )(s1, g0, dinv, Wc1, bc1)


def _dense2_body(s2_ref, g1_ref, dinv_ref, wmu2_ref, bmu2_ref, wlg2_ref,
                 blg2_ref, wmu3_ref, wlg3_ref, gt_ref):
    dinv = dinv_ref[...]
    a2 = (s2_ref[0] + s2_ref[1] + g1_ref[...]) * dinv
    h2mu = jax.nn.relu(jnp.dot(a2[:, :64], wmu2_ref[...],
                               preferred_element_type=jnp.float32)
                       + bmu2_ref[...])
    h2lg = jax.nn.relu(jnp.dot(a2[:, 64:], wlg2_ref[...],
                               preferred_element_type=jnp.float32)
                       + blg2_ref[...])
    tmu = jnp.dot(h2mu, wmu3_ref[...], preferred_element_type=jnp.float32)
    tlg = jnp.dot(h2lg, wlg3_ref[...], preferred_element_type=jnp.float32)
    gt_ref[...] = jnp.concatenate([tmu, tlg], axis=1) * dinv


def _dense2_call(s2, g1, dinv, Wmu2, bmu2, Wlg2, blg2, Wmu3, Wlg3):
    return pl.pallas_call(
        _dense2_body,
        grid=(N // BN,),
        in_specs=[_part_spec(128), _row_spec(128), _row_spec(1),
                  _full_spec(64, 128), _full_spec(1, 128),
                  _full_spec(64, 128), _full_spec(1, 128),
                  _full_spec(128, 32), _full_spec(128, 32)],
        out_specs=_row_spec(64),
        out_shape=jax.ShapeDtypeStruct((N, 64), jnp.float32),
    )(s2, g1, dinv, Wmu2, bmu2, Wlg2, blg2, Wmu3, Wlg3)


def _dense3_body(s3_ref, gt_ref, dinv_ref, bmu3_ref, blg3_ref, z_ref, ls_ref):
    a3 = (s3_ref[0] + s3_ref[1] + gt_ref[...]) * dinv_ref[...]
    z_ref[...] = jax.nn.relu(a3[:, :32] + bmu3_ref[...])
    ls_ref[...] = jnp.minimum(jax.nn.relu(a3[:, 32:] + blg3_ref[...]), 10.0)


def _dense3_call(s3, gt, dinv, bmu3, blg3):
    return pl.pallas_call(
        _dense3_body,
        grid=(N // BN,),
        in_specs=[_part_spec(64), _row_spec(64), _row_spec(1),
                  _full_spec(1, 32), _full_spec(1, 32)],
        out_specs=[_row_spec(32), _row_spec(32)],
        out_shape=[jax.ShapeDtypeStruct((N, 32), jnp.float32),
                   jax.ShapeDtypeStruct((N, 32), jnp.float32)],
    )(s3, gt, dinv, bmu3, blg3)


def kernel(x, edge_index, W_mu1, b_mu1, W_mu2, b_mu2, W_mu3, b_mu3,
           W_lg1, b_lg1, W_lg2, b_lg2, W_lg3, b_lg3):
    src2d = edge_index[0].reshape(R, CHUNK)
    dst2d = edge_index[1].reshape(R, CHUNK)

    Wc1 = jnp.concatenate([W_mu1, W_lg1], axis=1)
    bc1 = jnp.concatenate([b_mu1, b_lg1]).reshape(1, 128)

    degp = _deg_call(dst2d)
    degp = degp.reshape(NC, NPAD, 1)
    dinv, g0 = _scale0_call(degp, x)

    s1 = _seg_call(g0, src2d, dst2d)
    g1 = _dense1_call(s1, g0, dinv, Wc1, bc1)

    s2 = _seg_call(g1, src2d, dst2d)
    gt = _dense2_call(s2, g1, dinv, W_mu2, b_mu2.reshape(1, 128),
                      W_lg2, b_lg2.reshape(1, 128), W_mu3, W_lg3)

    s3 = _seg_call(gt, src2d, dst2d)
    z, logstd = _dense3_call(s3, gt, dinv, b_mu3.reshape(1, 32),
                             b_lg3.reshape(1, 32))

    adj = _dec_call(z, src2d, dst2d).reshape(E)
    return (adj, z, logstd)
